# baseline (speedup 1.0000x reference)
import jax
import jax.numpy as jnp
from jax.experimental import pallas as pl
from jax.experimental.pallas import tpu as pltpu

_NCH = 4


def _mask_dma_body(m_ref, x_any, o_any, zbuf, sem_c, sem_z):
    nslab = x_any.shape[0]
    ch = x_any.shape[1] // _NCH
    zbuf[...] = jnp.zeros_like(zbuf)

    def for_each(action):
        for i in range(nslab):
            m_on = m_ref[i] != 0
            for c in range(_NCH):
                sl = pl.ds(c * ch, ch)

                @pl.when(m_on)
                def _():
                    action(pltpu.make_async_copy(
                        x_any.at[i, sl], o_any.at[i, sl], sem_c))

                @pl.when(jnp.logical_not(m_on))
                def _():
                    action(pltpu.make_async_copy(
                        zbuf, o_any.at[i, sl], sem_z))

    for_each(lambda dma: dma.start())
    for_each(lambda dma: dma.wait())


def kernel(x, mask):
    B, K, C, H, W, Z = x.shape
    S = C * H * W * Z
    xf = x.reshape(B * K, S)
    m_i32 = mask.reshape(B * K).astype(jnp.int32)

    out = pl.pallas_call(
        _mask_dma_body,
        in_specs=[
            pl.BlockSpec(memory_space=pltpu.SMEM),
            pl.BlockSpec(memory_space=pl.ANY),
        ],
        out_specs=pl.BlockSpec(memory_space=pl.ANY),
        out_shape=jax.ShapeDtypeStruct((B * K, S), x.dtype),
        scratch_shapes=[
            pltpu.VMEM((S // _NCH,), x.dtype),
            pltpu.SemaphoreType.DMA,
            pltpu.SemaphoreType.DMA,
        ],
    )(m_i32, xf)
    return out.reshape(B, K * C, H, W, Z)


# speedup vs baseline: 41.6772x; 41.6772x over previous
import jax
import jax.numpy as jnp
from jax.experimental import pallas as pl
from jax.experimental.pallas import tpu as pltpu


def _mask_body(m_ref, x_ref, o_ref):
    b = pl.program_id(0)
    k = pl.program_id(1)
    K = pl.num_programs(1)
    m = m_ref[b * K + k]
    o_ref[...] = jnp.where(m != 0, x_ref[0], jnp.zeros_like(o_ref))


def kernel(x, mask):
    B, K, C, H, W, Z = x.shape
    CB = C // 4
    m_i32 = mask.reshape(B * K).astype(jnp.int32)

    out = pl.pallas_call(
        _mask_body,
        grid_spec=pltpu.PrefetchScalarGridSpec(
            num_scalar_prefetch=1,
            grid=(B, K, C // CB),
            in_specs=[pl.BlockSpec(
                (1, 1, CB, H, W, Z),
                lambda b, k, c, m: (b, k, c, 0, 0, 0))],
            out_specs=pl.BlockSpec(
                (1, CB, H, W, Z),
                lambda b, k, c, m: (b, k * (C // CB) + c, 0, 0, 0)),
        ),
        out_shape=jax.ShapeDtypeStruct((B, K * C, H, W, Z), x.dtype),
        compiler_params=pltpu.CompilerParams(
            dimension_semantics=("parallel", "parallel", "parallel")),
    )(m_i32, x)
    return out


# speedup vs baseline: 57.0197x; 1.3681x over previous
import jax
import jax.numpy as jnp
from jax.experimental import pallas as pl
from jax.experimental.pallas import tpu as pltpu


def _mask_body(m_ref, x_ref, o_ref):
    b = pl.program_id(0)
    k = pl.program_id(1)
    K = pl.num_programs(1)
    m = m_ref[b * K + k]
    o_ref[...] = jnp.where(m != 0, x_ref[0], jnp.zeros_like(o_ref))


def kernel(x, mask):
    B, K, C, H, W, Z = x.shape
    CB = C // 4
    m_i32 = mask.reshape(B * K).astype(jnp.int32)

    def x_map(b, k, c, m):
        on = m[b * K + k] != 0
        z = jnp.int32(0)
        return (jnp.where(on, b, z), jnp.where(on, k, z),
                jnp.where(on, c, z), 0, 0, 0)

    out = pl.pallas_call(
        _mask_body,
        grid_spec=pltpu.PrefetchScalarGridSpec(
            num_scalar_prefetch=1,
            grid=(B, K, C // CB),
            in_specs=[pl.BlockSpec((1, 1, CB, H, W, Z), x_map)],
            out_specs=pl.BlockSpec(
                (1, CB, H, W, Z),
                lambda b, k, c, m: (b, k * (C // CB) + c, 0, 0, 0)),
        ),
        out_shape=jax.ShapeDtypeStruct((B, K * C, H, W, Z), x.dtype),
        compiler_params=pltpu.CompilerParams(
            dimension_semantics=("parallel", "parallel", "parallel")),
    )(m_i32, x)
    return out


# speedup vs baseline: 58.0593x; 1.0182x over previous
import jax
import jax.numpy as jnp
from jax.experimental import pallas as pl
from jax.experimental.pallas import tpu as pltpu


def _mask_body(m_ref, al_ref, x_ref, o_ref):
    b = pl.program_id(0)
    k = pl.program_id(1)
    K = pl.num_programs(1)
    m = m_ref[b * K + k]
    o_ref[...] = jnp.where(m != 0, x_ref[0], jnp.zeros_like(o_ref))


def kernel(x, mask):
    B, K, C, H, W, Z = x.shape
    CB = C // 4
    m_i32 = mask.reshape(B * K).astype(jnp.int32)

    NC = C // CB
    s_idx = jnp.arange(B * K, dtype=jnp.int32)
    prev_on = jax.lax.cummax(jnp.where(m_i32 != 0, s_idx, -1))
    alias_flat = jnp.where(prev_on >= 0, prev_on * NC + (NC - 1), 0)

    def x_map(b, k, c, m, al):
        s = b * K + k
        flat = jnp.where(m[s] != 0, s * NC + c, al[s])
        return (flat // (K * NC), (flat // NC) % K, flat % NC, 0, 0, 0)

    out = pl.pallas_call(
        _mask_body,
        grid_spec=pltpu.PrefetchScalarGridSpec(
            num_scalar_prefetch=2,
            grid=(B, K, C // CB),
            in_specs=[pl.BlockSpec((1, 1, CB, H, W, Z), x_map)],
            out_specs=pl.BlockSpec(
                (1, CB, H, W, Z),
                lambda b, k, c, m, al: (b, k * (C // CB) + c, 0, 0, 0)),
        ),
        out_shape=jax.ShapeDtypeStruct((B, K * C, H, W, Z), x.dtype),
        compiler_params=pltpu.CompilerParams(
            dimension_semantics=("parallel", "parallel", "parallel")),
    )(m_i32, alias_flat, x)
    return out


# speedup vs baseline: 58.3719x; 1.0054x over previous
import jax
import jax.numpy as jnp
from jax.experimental import pallas as pl
from jax.experimental.pallas import tpu as pltpu


def _mask_body(m_ref, al_ref, x_ref, o_ref):
    b = pl.program_id(0)
    k = pl.program_id(1)
    K = pl.num_programs(1)
    m = m_ref[b * K + k]
    o_ref[...] = jnp.where(m != 0, x_ref[0], jnp.zeros_like(o_ref))


def kernel(x, mask):
    B, K, C, H, W, Z = x.shape
    CB = C // 4
    m_i32 = mask.reshape(B * K).astype(jnp.int32)

    NC = C // CB
    s_idx = jnp.arange(B * K, dtype=jnp.int32)
    on = m_i32 != 0
    prev_on = jax.lax.cummax(jnp.where(on, s_idx, -1))
    first_on = jnp.where(jnp.any(on), jnp.argmax(on).astype(jnp.int32), 0)
    alias_flat = jnp.where(prev_on >= 0, prev_on * NC + (NC - 1),
                           first_on * NC)

    def x_map(b, k, c, m, al):
        s = b * K + k
        flat = jnp.where(m[s] != 0, s * NC + c, al[s])
        return (flat // (K * NC), (flat // NC) % K, flat % NC, 0, 0, 0)

    out = pl.pallas_call(
        _mask_body,
        grid_spec=pltpu.PrefetchScalarGridSpec(
            num_scalar_prefetch=2,
            grid=(B, K, C // CB),
            in_specs=[pl.BlockSpec((1, 1, CB, H, W, Z), x_map)],
            out_specs=pl.BlockSpec(
                (1, CB, H, W, Z),
                lambda b, k, c, m, al: (b, k * (C // CB) + c, 0, 0, 0)),
        ),
        out_shape=jax.ShapeDtypeStruct((B, K * C, H, W, Z), x.dtype),
        compiler_params=pltpu.CompilerParams(
            dimension_semantics=("parallel", "parallel", "parallel")),
    )(m_i32, alias_flat, x)
    return out
